# Initial kernel scaffold; baseline (speedup 1.0000x reference)
import jax
import jax.numpy as jnp
from jax.experimental import pallas as pl


def kernel(seq, table):
    raise NotImplementedError("write your pallas kernel here")


# speedup vs baseline: 28.2674x; 28.2674x over previous
import functools

import jax
import jax.numpy as jnp
from jax import lax
from jax.experimental import pallas as pl
from jax.experimental.pallas import tpu as pltpu
from jax.experimental.pallas import tpu_sc as plsc

_L = 16


def _make_sc_kernel(n, block):
    num_blocks = n // block
    assert n % block == 0 and block % _L == 0
    mesh = plsc.VectorSubcoreMesh(core_axis_name="c", subcore_axis_name="s")
    info = plsc.get_sparse_core_info()
    nw = info.num_cores * info.num_subcores

    @functools.partial(
        pl.kernel,
        mesh=mesh,
        out_type=jax.ShapeDtypeStruct((4 * n,), jnp.float32),
        compiler_params=pltpu.CompilerParams(needs_layout_passes=False),
        scratch_types=[
            pltpu.VMEM((24,), jnp.float32),
            pltpu.VMEM((block,), jnp.int32),
            pltpu.VMEM((block,), jnp.float32),
            pltpu.VMEM((block,), jnp.float32),
            pltpu.VMEM((block,), jnp.float32),
            pltpu.VMEM((block,), jnp.float32),
        ],
    )
    def sc_kernel(seq_hbm, table_hbm, out_hbm, tab_v, seq_v, o0, o1, o2, o3):
        wid = lax.axis_index("s") * info.num_cores + lax.axis_index("c")
        pltpu.sync_copy(table_hbm, tab_v)

        outs = (o0, o1, o2, o3)

        def do_block(blk, _):
            start = blk * block
            pltpu.sync_copy(seq_hbm.at[pl.ds(start, block)], seq_v)

            def do_vec(i, _):
                off = i * _L
                idx4 = seq_v[pl.ds(off, _L)] * 4
                for c in range(4):
                    vals = plsc.load_gather(tab_v, [idx4 + c])
                    outs[c][pl.ds(off, _L)] = vals
                return 0

            lax.fori_loop(0, block // _L, do_vec, 0, unroll=4)
            for c in range(4):
                pltpu.sync_copy(outs[c], out_hbm.at[pl.ds(c * n + start, block)])
            return 0

        nblk = (num_blocks - wid + nw - 1) // nw
        lax.fori_loop(0, nblk, lambda k, _: do_block(wid + k * nw, _), 0)

    return sc_kernel


@jax.jit
def kernel(seq, table):
    n = seq.shape[0]
    block = 8000
    tab_flat = jnp.concatenate(
        [table.reshape(-1), jnp.zeros((4,), dtype=table.dtype)])
    flat = _make_sc_kernel(n, block)(seq, tab_flat)
    return flat.reshape(4, n)


# speedup vs baseline: 30.9595x; 1.0952x over previous
import functools

import jax
import jax.numpy as jnp
from jax import lax
from jax.experimental import pallas as pl
from jax.experimental.pallas import tpu as pltpu
from jax.experimental.pallas import tpu_sc as plsc

_L = 16


def _make_sc_kernel(n, block):
    num_blocks = n // block
    assert n % block == 0 and block % _L == 0
    mesh = plsc.VectorSubcoreMesh(core_axis_name="c", subcore_axis_name="s")
    info = plsc.get_sparse_core_info()
    nw = info.num_cores * info.num_subcores
    kmax = -(-num_blocks // nw)
    n_long = num_blocks - (kmax - 1) * nw

    @functools.partial(
        pl.kernel,
        mesh=mesh,
        out_type=jax.ShapeDtypeStruct((4 * n,), jnp.float32),
        compiler_params=pltpu.CompilerParams(needs_layout_passes=False),
        scratch_types=[
            pltpu.VMEM((32,), jnp.float32),
            pltpu.VMEM((block,), jnp.int32),
            pltpu.VMEM((block,), jnp.int32),
            pltpu.VMEM((block,), jnp.float32),
            pltpu.VMEM((block,), jnp.float32),
            pltpu.VMEM((block,), jnp.float32),
            pltpu.VMEM((block,), jnp.float32),
            pltpu.VMEM((block,), jnp.float32),
            pltpu.VMEM((block,), jnp.float32),
            pltpu.VMEM((block,), jnp.float32),
            pltpu.VMEM((block,), jnp.float32),
            pltpu.SemaphoreType.DMA,
            pltpu.SemaphoreType.DMA,
            pltpu.SemaphoreType.DMA,
            pltpu.SemaphoreType.DMA,
        ],
    )
    def sc_kernel(seq_hbm, tab_hbm, out_hbm, tab_v,
                  s0, s1, a0, a1, a2, a3, b0, b1, b2, b3,
                  si0, si1, so0, so1):
        wid = lax.axis_index("s") * info.num_cores + lax.axis_index("c")
        seq_bufs = (s0, s1)
        out_bufs = ((a0, a1, a2, a3), (b0, b1, b2, b3))
        sem_in = (si0, si1)
        sem_out = (so0, so1)

        pltpu.sync_copy(tab_hbm, tab_v)

        def start(k):
            return (wid + k * nw) * block

        def seq_copy(k):
            return pltpu.make_async_copy(
                seq_hbm.at[pl.ds(start(k), block)],
                seq_bufs[k % 2], sem_in[k % 2])

        def out_copy(k, c):
            return pltpu.make_async_copy(
                out_bufs[k % 2][c],
                out_hbm.at[pl.ds(c * n + start(k), block)],
                sem_out[k % 2])

        def compute(k):
            sbuf = seq_bufs[k % 2]
            obufs = out_bufs[k % 2]

            def do_vec(i, _):
                off = i * _L
                idx = sbuf[pl.ds(off, _L)]
                for c in range(4):
                    gidx = idx if c == 0 else idx + (c * 8)
                    obufs[c][pl.ds(off, _L)] = plsc.load_gather(tab_v, [gidx])
                return 0

            lax.fori_loop(0, block // _L, do_vec, 0, unroll=8)

        is_long = wid < n_long

        seq_copy(0).start()
        seq_copy(1).start()
        for k in range(kmax):
            def step(k=k):
                seq_copy(k).wait()
                if k >= 2:
                    for c in range(4):
                        out_copy(k - 2, c).wait()
                compute(k)
                for c in range(4):
                    out_copy(k, c).start()
                if k + 2 < kmax:
                    if k + 2 == kmax - 1:
                        pl.when(is_long)(lambda: seq_copy(k + 2).start())
                    else:
                        seq_copy(k + 2).start()

            if k == kmax - 1:
                pl.when(is_long)(step)
            else:
                step()

        def drain(k):
            for c in range(4):
                out_copy(k, c).wait()

        drain(kmax - 2)
        pl.when(is_long)(lambda: drain(kmax - 1))
        pl.when(jnp.logical_not(is_long))(lambda: drain(kmax - 3))

    return sc_kernel


@jax.jit
def kernel(seq, table):
    n = seq.shape[0]
    block = 8000
    tab_t = jnp.pad(table.T, ((0, 0), (0, 3))).reshape(-1)
    flat = _make_sc_kernel(n, block)(seq, tab_t)
    return flat.reshape(4, n)


# speedup vs baseline: 49.9792x; 1.6143x over previous
import functools

import jax
import jax.numpy as jnp
from jax import lax
from jax.experimental import pallas as pl
from jax.experimental.pallas import tpu as pltpu
from jax.experimental.pallas import tpu_sc as plsc

_L = 16


def _make_sc_kernel(n, block):
    num_blocks = n // block
    assert n % block == 0 and block % _L == 0
    mesh = plsc.VectorSubcoreMesh(core_axis_name="c", subcore_axis_name="s")
    info = plsc.get_sparse_core_info()
    nw = info.num_cores * info.num_subcores
    kmax = -(-num_blocks // nw)
    n_long = num_blocks - (kmax - 1) * nw

    @functools.partial(
        pl.kernel,
        mesh=mesh,
        out_type=jax.ShapeDtypeStruct((4 * n,), jnp.float32),
        compiler_params=pltpu.CompilerParams(needs_layout_passes=False),
        scratch_types=[
            pltpu.VMEM((32,), jnp.float32),
            pltpu.VMEM((block,), jnp.int32),
            pltpu.VMEM((block,), jnp.int32),
            pltpu.VMEM((block,), jnp.float32),
            pltpu.VMEM((block,), jnp.float32),
            pltpu.VMEM((block,), jnp.float32),
            pltpu.VMEM((block,), jnp.float32),
            pltpu.VMEM((block,), jnp.float32),
            pltpu.VMEM((block,), jnp.float32),
            pltpu.VMEM((block,), jnp.float32),
            pltpu.VMEM((block,), jnp.float32),
            pltpu.SemaphoreType.DMA,
            pltpu.SemaphoreType.DMA,
            pltpu.SemaphoreType.DMA,
            pltpu.SemaphoreType.DMA,
        ],
    )
    def sc_kernel(seq_hbm, tab_hbm, out_hbm, tab_v,
                  s0, s1, a0, a1, a2, a3, b0, b1, b2, b3,
                  si0, si1, so0, so1):
        wid = lax.axis_index("s") * info.num_cores + lax.axis_index("c")
        seq_bufs = (s0, s1)
        out_bufs = ((a0, a1, a2, a3), (b0, b1, b2, b3))
        sem_in = (si0, si1)
        sem_out = (so0, so1)

        pltpu.sync_copy(tab_hbm, tab_v)

        def start(k):
            return (wid + k * nw) * block

        def seq_copy(k):
            return pltpu.make_async_copy(
                seq_hbm.at[pl.ds(start(k), block)],
                seq_bufs[k % 2], sem_in[k % 2])

        def out_copy(k, c):
            return pltpu.make_async_copy(
                out_bufs[k % 2][c],
                out_hbm.at[pl.ds(c * n + start(k), block)],
                sem_out[k % 2])

        def compute(k):
            sbuf = seq_bufs[k % 2]
            obufs = out_bufs[k % 2]

            @plsc.parallel_loop(0, block, _L, unroll=8)
            def _(off):
                idx = sbuf[pl.ds(off, _L)]
                for c in range(4):
                    gidx = idx if c == 0 else idx + (c * 8)
                    obufs[c][pl.ds(off, _L)] = plsc.load_gather(tab_v, [gidx])

        is_long = wid < n_long

        seq_copy(0).start()
        seq_copy(1).start()
        for k in range(kmax):
            def step(k=k):
                seq_copy(k).wait()
                if k >= 2:
                    for c in range(4):
                        out_copy(k - 2, c).wait()
                compute(k)
                for c in range(4):
                    out_copy(k, c).start()
                if k + 2 < kmax:
                    if k + 2 == kmax - 1:
                        pl.when(is_long)(lambda: seq_copy(k + 2).start())
                    else:
                        seq_copy(k + 2).start()

            if k == kmax - 1:
                pl.when(is_long)(step)
            else:
                step()

        def drain(k):
            for c in range(4):
                out_copy(k, c).wait()

        drain(kmax - 2)
        pl.when(is_long)(lambda: drain(kmax - 1))
        pl.when(jnp.logical_not(is_long))(lambda: drain(kmax - 3))

    return sc_kernel


@jax.jit
def kernel(seq, table):
    n = seq.shape[0]
    block = 8000
    tab_t = jnp.pad(table.T, ((0, 0), (0, 3))).reshape(-1)
    flat = _make_sc_kernel(n, block)(seq, tab_t)
    return flat.reshape(4, n)


# speedup vs baseline: 82.7151x; 1.6550x over previous
import functools

import jax
import jax.numpy as jnp
from jax import lax
from jax.experimental import pallas as pl
from jax.experimental.pallas import tpu as pltpu
from jax.experimental.pallas import tpu_sc as plsc

_L = 16


def _make_sc_kernel(n, block):
    nbf = n // block
    tail = n - nbf * block
    assert block % 128 == 0 and tail % _L == 0
    mesh = plsc.VectorSubcoreMesh(core_axis_name="c", subcore_axis_name="s")
    info = plsc.get_sparse_core_info()
    nw = info.num_cores * info.num_subcores
    kmax = -(-nbf // nw)
    n_long = nbf - (kmax - 1) * nw

    @functools.partial(
        pl.kernel,
        mesh=mesh,
        out_type=jax.ShapeDtypeStruct((4, n), jnp.float32),
        compiler_params=pltpu.CompilerParams(needs_layout_passes=False),
        scratch_types=[
            pltpu.VMEM((32,), jnp.float32),
            pltpu.VMEM((block,), jnp.int32),
            pltpu.VMEM((block,), jnp.int32),
            pltpu.VMEM((4, block), jnp.float32),
            pltpu.VMEM((4, block), jnp.float32),
            pltpu.VMEM((max(tail, _L),), jnp.int32),
            pltpu.VMEM((4, max(tail, _L)), jnp.float32),
            pltpu.SemaphoreType.DMA,
            pltpu.SemaphoreType.DMA,
            pltpu.SemaphoreType.DMA,
            pltpu.SemaphoreType.DMA,
        ],
    )
    def sc_kernel(seq_hbm, tab_hbm, out_hbm, tab_v,
                  s0, s1, ob0, ob1, st, ot, si0, si1, so0, so1):
        wid = lax.axis_index("s") * info.num_cores + lax.axis_index("c")
        seq_bufs = (s0, s1)
        out_bufs = (ob0, ob1)
        sem_in = (si0, si1)
        sem_out = (so0, so1)

        pltpu.sync_copy(tab_hbm, tab_v)

        def start(k):
            return (wid + k * nw) * block

        def seq_copy(k):
            return pltpu.make_async_copy(
                seq_hbm.at[pl.ds(start(k), block)],
                seq_bufs[k % 2], sem_in[k % 2])

        def out_copy(k):
            return pltpu.make_async_copy(
                out_bufs[k % 2],
                out_hbm.at[:, pl.ds(start(k), block)],
                sem_out[k % 2])

        def gather_rows(obuf, idx, off):
            for c in range(4):
                gidx = idx if c == 0 else idx + (c * 8)
                obuf[c, pl.ds(off, _L)] = plsc.load_gather(tab_v, [gidx])

        def compute(k):
            sbuf = seq_bufs[k % 2]
            obuf = out_bufs[k % 2]

            @plsc.parallel_loop(0, block, _L, unroll=8)
            def _(off):
                gather_rows(obuf, sbuf[pl.ds(off, _L)], off)

        is_long = wid < n_long

        seq_copy(0).start()
        seq_copy(1).start()
        for k in range(kmax):
            def step(k=k):
                seq_copy(k).wait()
                if k >= 2:
                    out_copy(k - 2).wait()
                compute(k)
                out_copy(k).start()
                if k + 2 < kmax:
                    if k + 2 == kmax - 1:
                        pl.when(is_long)(lambda: seq_copy(k + 2).start())
                    else:
                        seq_copy(k + 2).start()

            if k == kmax - 1:
                pl.when(is_long)(step)
            else:
                step()

        if tail:
            @pl.when(wid == nw - 1)
            def _():
                tstart = nbf * block
                pltpu.sync_copy(seq_hbm.at[pl.ds(tstart, tail)], st)
                for j in range(tail // _L):
                    gather_rows(ot, st[pl.ds(j * _L, _L)], j * _L)
                pltpu.sync_copy(ot, out_hbm.at[:, pl.ds(tstart, tail)])

        out_copy(kmax - 2).wait()
        pl.when(is_long)(lambda: out_copy(kmax - 1).wait())
        pl.when(jnp.logical_not(is_long))(lambda: out_copy(kmax - 3).wait())

    return sc_kernel


@jax.jit
def kernel(seq, table):
    n = seq.shape[0]
    block = 8064
    tab_t = jnp.pad(table.T, ((0, 0), (0, 3))).reshape(-1)
    return _make_sc_kernel(n, block)(seq, tab_t)


# speedup vs baseline: 82.8490x; 1.0016x over previous
import functools

import jax
import jax.numpy as jnp
from jax import lax
from jax.experimental import pallas as pl
from jax.experimental.pallas import tpu as pltpu
from jax.experimental.pallas import tpu_sc as plsc

_L = 16


def _make_sc_kernel(n, block):
    nbf = n // block
    tail = n - nbf * block
    assert block % 128 == 0 and tail % _L == 0
    mesh = plsc.VectorSubcoreMesh(core_axis_name="c", subcore_axis_name="s")
    info = plsc.get_sparse_core_info()
    nw = info.num_cores * info.num_subcores
    assert nbf % nw == 0
    kmax = nbf // nw

    @functools.partial(
        pl.kernel,
        mesh=mesh,
        out_type=jax.ShapeDtypeStruct((4, n), jnp.float32),
        compiler_params=pltpu.CompilerParams(needs_layout_passes=False),
        scratch_types=[
            pltpu.VMEM((20,), jnp.float32),
            pltpu.VMEM((block,), jnp.int32),
            pltpu.VMEM((block,), jnp.int32),
            pltpu.VMEM((4, block), jnp.float32),
            pltpu.VMEM((4, block), jnp.float32),
            pltpu.VMEM((max(tail, _L),), jnp.int32),
            pltpu.VMEM((4, max(tail, _L)), jnp.float32),
            pltpu.SemaphoreType.DMA,
            pltpu.SemaphoreType.DMA,
            pltpu.SemaphoreType.DMA,
            pltpu.SemaphoreType.DMA,
        ],
    )
    def sc_kernel(seq_hbm, tab_hbm, out_hbm, tab_v,
                  s0, s1, ob0, ob1, st, ot, si0, si1, so0, so1):
        wid = lax.axis_index("s") * info.num_cores + lax.axis_index("c")
        seq_bufs = (s0, s1)
        out_bufs = (ob0, ob1)
        sem_in = (si0, si1)
        sem_out = (so0, so1)

        pltpu.sync_copy(tab_hbm, tab_v)

        def start(k):
            return (wid + k * nw) * block

        def seq_copy(k):
            return pltpu.make_async_copy(
                seq_hbm.at[pl.ds(start(k), block)],
                seq_bufs[k % 2], sem_in[k % 2])

        def out_copy(k):
            return pltpu.make_async_copy(
                out_bufs[k % 2],
                out_hbm.at[:, pl.ds(start(k), block)],
                sem_out[k % 2])

        def gather_rows(obuf, idx, off):
            idx4 = idx * 4
            for c in range(4):
                gidx = idx4 if c == 0 else idx4 + c
                obuf[c, pl.ds(off, _L)] = plsc.load_gather(tab_v, [gidx])

        def compute(k):
            sbuf = seq_bufs[k % 2]
            obuf = out_bufs[k % 2]

            @plsc.parallel_loop(0, block, _L, unroll=8)
            def _(off):
                gather_rows(obuf, sbuf[pl.ds(off, _L)], off)

        seq_copy(0).start()
        if kmax > 1:
            seq_copy(1).start()
        for k in range(kmax):
            seq_copy(k).wait()
            if k >= 2:
                out_copy(k - 2).wait()
            compute(k)
            out_copy(k).start()
            if k + 2 < kmax:
                seq_copy(k + 2).start()

        if tail:
            @pl.when(wid == nw - 1)
            def _():
                tstart = nbf * block
                pltpu.sync_copy(seq_hbm.at[pl.ds(tstart, tail)], st)

                @plsc.parallel_loop(0, tail, _L, unroll=4)
                def _(off):
                    gather_rows(ot, st[pl.ds(off, _L)], off)

                pltpu.sync_copy(
                    ot.at[:, pl.ds(0, tail)] if tail != max(tail, _L) else ot,
                    out_hbm.at[:, pl.ds(tstart, tail)])

        if kmax > 1:
            out_copy(kmax - 2).wait()
        out_copy(kmax - 1).wait()

    return sc_kernel


@jax.jit
def kernel(seq, table):
    n = seq.shape[0]
    block = 7808
    return _make_sc_kernel(n, block)(seq, table.reshape(-1))


# speedup vs baseline: 83.2410x; 1.0047x over previous
import functools

import jax
import jax.numpy as jnp
from jax import lax
from jax.experimental import pallas as pl
from jax.experimental.pallas import tpu as pltpu
from jax.experimental.pallas import tpu_sc as plsc

_L = 16


def _make_sc_kernel(n, block):
    nbf = n // block
    tail = n - nbf * block
    assert block % 128 == 0 and tail % _L == 0
    mesh = plsc.VectorSubcoreMesh(core_axis_name="c", subcore_axis_name="s")
    info = plsc.get_sparse_core_info()
    nw = info.num_cores * info.num_subcores
    assert nbf % nw == 0
    kmax = nbf // nw

    @functools.partial(
        pl.kernel,
        mesh=mesh,
        out_type=jax.ShapeDtypeStruct((4, n), jnp.float32),
        compiler_params=pltpu.CompilerParams(
            needs_layout_passes=False, skip_device_barrier=True),
        scratch_types=[
            pltpu.VMEM((20,), jnp.float32),
            pltpu.VMEM((block,), jnp.int32),
            pltpu.VMEM((block,), jnp.int32),
            pltpu.VMEM((4, block), jnp.float32),
            pltpu.VMEM((4, block), jnp.float32),
            pltpu.VMEM((max(tail, _L),), jnp.int32),
            pltpu.VMEM((4, max(tail, _L)), jnp.float32),
            pltpu.SemaphoreType.DMA,
            pltpu.SemaphoreType.DMA,
            pltpu.SemaphoreType.DMA,
            pltpu.SemaphoreType.DMA,
        ],
    )
    def sc_kernel(seq_hbm, tab_hbm, out_hbm, tab_v,
                  s0, s1, ob0, ob1, st, ot, si0, si1, so0, so1):
        wid = lax.axis_index("s") * info.num_cores + lax.axis_index("c")
        seq_bufs = (s0, s1)
        out_bufs = (ob0, ob1)
        sem_in = (si0, si1)
        sem_out = (so0, so1)

        pltpu.sync_copy(tab_hbm, tab_v)

        def start(k):
            return (wid + k * nw) * block

        def seq_copy(k):
            return pltpu.make_async_copy(
                seq_hbm.at[pl.ds(start(k), block)],
                seq_bufs[k % 2], sem_in[k % 2])

        def out_copy(k):
            return pltpu.make_async_copy(
                out_bufs[k % 2],
                out_hbm.at[:, pl.ds(start(k), block)],
                sem_out[k % 2])

        def gather_rows(obuf, idx, off):
            idx4 = idx * 4
            for c in range(4):
                gidx = idx4 if c == 0 else idx4 + c
                obuf[c, pl.ds(off, _L)] = plsc.load_gather(tab_v, [gidx])

        def compute(k):
            sbuf = seq_bufs[k % 2]
            obuf = out_bufs[k % 2]

            @plsc.parallel_loop(0, block, _L, unroll=8)
            def _(off):
                gather_rows(obuf, sbuf[pl.ds(off, _L)], off)

        seq_copy(0).start()
        if kmax > 1:
            seq_copy(1).start()
        for k in range(kmax):
            seq_copy(k).wait()
            if k >= 2:
                out_copy(k - 2).wait()
            compute(k)
            out_copy(k).start()
            if k + 2 < kmax:
                seq_copy(k + 2).start()

        if tail:
            @pl.when(wid == nw - 1)
            def _():
                tstart = nbf * block
                pltpu.sync_copy(seq_hbm.at[pl.ds(tstart, tail)], st)

                @plsc.parallel_loop(0, tail, _L, unroll=4)
                def _(off):
                    gather_rows(ot, st[pl.ds(off, _L)], off)

                pltpu.sync_copy(
                    ot.at[:, pl.ds(0, tail)] if tail != max(tail, _L) else ot,
                    out_hbm.at[:, pl.ds(tstart, tail)])

        if kmax > 1:
            out_copy(kmax - 2).wait()
        out_copy(kmax - 1).wait()

    return sc_kernel


@jax.jit
def kernel(seq, table):
    n = seq.shape[0]
    block = 7808
    return _make_sc_kernel(n, block)(seq, table.reshape(-1))


# speedup vs baseline: 83.5363x; 1.0035x over previous
import functools

import jax
import jax.numpy as jnp
from jax import lax
from jax.experimental import pallas as pl
from jax.experimental.pallas import tpu as pltpu
from jax.experimental.pallas import tpu_sc as plsc

_L = 16


def _make_sc_kernel(n, sizes, tail):
    mesh = plsc.VectorSubcoreMesh(core_axis_name="c", subcore_axis_name="s")
    info = plsc.get_sparse_core_info()
    nw = info.num_cores * info.num_subcores
    assert all(s % 128 == 0 for s in sizes) and tail % _L == 0
    assert nw * sum(sizes) + tail == n
    kmax = len(sizes)
    bmax = max(sizes)
    prefix = [nw * sum(sizes[:r]) for r in range(kmax)]

    @functools.partial(
        pl.kernel,
        mesh=mesh,
        out_type=jax.ShapeDtypeStruct((4, n), jnp.float32),
        compiler_params=pltpu.CompilerParams(
            needs_layout_passes=False, skip_device_barrier=True),
        scratch_types=[
            pltpu.VMEM((20,), jnp.float32),
            pltpu.VMEM((bmax,), jnp.int32),
            pltpu.VMEM((bmax,), jnp.int32),
            pltpu.VMEM((4, bmax), jnp.float32),
            pltpu.VMEM((4, bmax), jnp.float32),
            pltpu.VMEM((max(tail, _L),), jnp.int32),
            pltpu.VMEM((4, max(tail, _L)), jnp.float32),
            pltpu.SemaphoreType.DMA,
            pltpu.SemaphoreType.DMA,
            pltpu.SemaphoreType.DMA,
            pltpu.SemaphoreType.DMA,
        ],
    )
    def sc_kernel(seq_hbm, tab_hbm, out_hbm, tab_v,
                  s0, s1, ob0, ob1, st, ot, si0, si1, so0, so1):
        wid = lax.axis_index("s") * info.num_cores + lax.axis_index("c")
        seq_bufs = (s0, s1)
        out_bufs = (ob0, ob1)
        sem_in = (si0, si1)
        sem_out = (so0, so1)

        def start(k):
            return prefix[k] + wid * sizes[k]

        def seq_copy(k):
            return pltpu.make_async_copy(
                seq_hbm.at[pl.ds(start(k), sizes[k])],
                seq_bufs[k % 2].at[pl.ds(0, sizes[k])], sem_in[k % 2])

        def out_copy(k):
            return pltpu.make_async_copy(
                out_bufs[k % 2].at[:, pl.ds(0, sizes[k])],
                out_hbm.at[:, pl.ds(start(k), sizes[k])],
                sem_out[k % 2])

        def gather_rows(obuf, idx, off):
            idx4 = idx * 4
            for c in range(4):
                gidx = idx4 if c == 0 else idx4 + c
                obuf[c, pl.ds(off, _L)] = plsc.load_gather(tab_v, [gidx])

        def compute(k):
            sbuf = seq_bufs[k % 2]
            obuf = out_bufs[k % 2]

            @plsc.parallel_loop(0, sizes[k], _L, unroll=8)
            def _(off):
                gather_rows(obuf, sbuf[pl.ds(off, _L)], off)

        seq_copy(0).start()
        if kmax > 1:
            seq_copy(1).start()
        pltpu.sync_copy(tab_hbm, tab_v)
        for k in range(kmax):
            seq_copy(k).wait()
            if k >= 2:
                out_copy(k - 2).wait()
            compute(k)
            out_copy(k).start()
            if k + 2 < kmax:
                seq_copy(k + 2).start()

        if tail:
            @pl.when(wid == nw - 1)
            def _():
                tstart = n - tail
                pltpu.sync_copy(seq_hbm.at[pl.ds(tstart, tail)], st)

                @plsc.parallel_loop(0, tail, _L, unroll=4)
                def _(off):
                    gather_rows(ot, st[pl.ds(off, _L)], off)

                pltpu.sync_copy(
                    ot.at[:, pl.ds(0, tail)] if tail != max(tail, _L) else ot,
                    out_hbm.at[:, pl.ds(tstart, tail)])

        if kmax > 1:
            out_copy(kmax - 2).wait()
        out_copy(kmax - 1).wait()

    return sc_kernel


@jax.jit
def kernel(seq, table):
    n = seq.shape[0]
    sizes = (1024, 8576, 8576, 8576, 3456, 1024)
    return _make_sc_kernel(n, sizes, n - 32 * sum(sizes))(seq, table.reshape(-1))
